# Initial kernel scaffold; baseline (speedup 1.0000x reference)
import jax
import jax.numpy as jnp
from jax.experimental import pallas as pl


def kernel(x, edge_index, edge_attr, batch, W1, b1, W2, b2, W3, b3):
    raise NotImplementedError("write your pallas kernel here")


# speedup vs baseline: 4.2600x; 4.2600x over previous
import jax
import jax.numpy as jnp
from jax import lax
from jax.experimental import pallas as pl
from jax.experimental.pallas import tpu as pltpu
from jax.experimental.pallas import tpu_sc as plsc

_N = 10000
_E = 160000
_D = 256
_G = 64
_NC = 2
_NS = 16
_NW = _NC * _NS

_ER = 1280
_EP = _ER * 128
_EPAD_DST = 20000

_RNG = 320
_DUMP = _RNG
_ACC_ROWS = _RNG + 8
_CAP = 8192
_CHUNK = 64
_HLIST = 4096

_SCAN_BLK = 40

_mesh = plsc.VectorSubcoreMesh(
    core_axis_name="c", subcore_axis_name="s", num_cores=_NC, num_subcores=_NS
)
_sc_params = pltpu.CompilerParams(needs_layout_passes=False)

def _tile_id():
    return lax.axis_index("s") * _NC + lax.axis_index("c")


def _sc_prep_body(src_hbm, dst_hbm, elsrc_hbm, eldst_hbm, cnts_hbm, hist_hbm,
                  sb0, db0, sb1, db1, csrc, cldst, hist2d, cbuf, esem0, esem1):
    tg = _tile_id()
    lo = tg * _RNG
    zv = jnp.zeros((16,), jnp.float32)
    zi = jnp.zeros((16,), jnp.int32)

    def zh(i, _):
        for j in range(8):
            hist2d[i, pl.ds(j * 16, 16)] = zv
        return 0

    lax.fori_loop(0, 40, zh, 0)

    sbufs = (sb0, sb1)
    dbufs = (db0, db1)
    esems = (esem0, esem1)

    def fetch(b, p):
        pltpu.async_copy(src_hbm.at[pl.ds(b * _SCAN_BLK, _SCAN_BLK)],
                         sbufs[p], esems[p])
        pltpu.async_copy(dst_hbm.at[pl.ds(b * _SCAN_BLK, _SCAN_BLK)],
                         dbufs[p], esems[p])

    def fwait(p):
        pltpu.make_async_copy(src_hbm.at[pl.ds(0, _SCAN_BLK)], sbufs[p],
                              esems[p]).wait()
        pltpu.make_async_copy(dst_hbm.at[pl.ds(0, _SCAN_BLK)], dbufs[p],
                              esems[p]).wait()

    lane = lax.iota(jnp.int32, 16)
    onesf = jnp.ones((16,), jnp.float32)
    nblk = _ER // _SCAN_BLK

    def scan_block(p, cnt):
        def sg(k, cnt):
            j = k // 8
            q = (k % 8) * 16
            dvec = dbufs[p][j, pl.ds(q, 16)]
            svec = sbufs[p][j, pl.ds(q, 16)]
            ld = dvec - lo
            m = (dvec >= lo) & (dvec < lo + _RNG)
            plsc.addupdate_scatter(
                hist2d, [ld >> 3, ((ld & 7) << 4) + lane], onesf, mask=m)
            plsc.store_compressed(csrc.at[pl.ds(cnt, 16)], svec, mask=m)
            plsc.store_compressed(cldst.at[pl.ds(cnt, 16)], ld, mask=m)
            return cnt + jnp.max(plsc.all_reduce_population_count(m))

        return lax.fori_loop(0, _SCAN_BLK * 8, sg, cnt)

    fetch(0, 0)
    fetch(1, 1)

    def blk_loop(k, cnt):
        b = 2 * k
        fwait(0)
        cnt = scan_block(0, cnt)

        @pl.when(b + 2 < nblk)
        def _():
            fetch(b + 2, 0)

        fwait(1)
        cnt = scan_block(1, cnt)

        @pl.when(b + 3 < nblk)
        def _():
            fetch(b + 3, 1)

        return cnt

    cnt = lax.fori_loop(0, nblk // 2, blk_loop, jnp.int32(0))

    dumpv = jnp.full((16,), _DUMP, jnp.int32)
    for q in range(4):
        csrc[pl.ds(cnt + q * 16, 16)] = zi
        cldst[pl.ds(cnt + q * 16, 16)] = dumpv
    cnt_pad = ((cnt + 63) // 64) * 64
    cbuf[pl.ds(0, 16)] = jnp.full((16,), cnt_pad, jnp.int32)

    pltpu.sync_copy(csrc, elsrc_hbm.at[tg])
    pltpu.sync_copy(cldst, eldst_hbm.at[tg])
    pltpu.sync_copy(cbuf, cnts_hbm.at[tg])
    pltpu.sync_copy(hist2d, hist_hbm.at[tg])


def _make_sc_prep():
    return pl.kernel(
        _sc_prep_body,
        out_type=(
            jax.ShapeDtypeStruct((_NW, _CAP), jnp.int32),
            jax.ShapeDtypeStruct((_NW, _CAP), jnp.int32),
            jax.ShapeDtypeStruct((_NW, 16), jnp.int32),
            jax.ShapeDtypeStruct((_NW, 40, 128), jnp.float32),
        ),
        mesh=_mesh,
        compiler_params=_sc_params,
        scratch_types=[
            pltpu.VMEM((_SCAN_BLK, 128), jnp.int32),
            pltpu.VMEM((_SCAN_BLK, 128), jnp.int32),
            pltpu.VMEM((_SCAN_BLK, 128), jnp.int32),
            pltpu.VMEM((_SCAN_BLK, 128), jnp.int32),
            pltpu.VMEM((_CAP,), jnp.int32),
            pltpu.VMEM((_CAP,), jnp.int32),
            pltpu.VMEM((40, 128), jnp.float32),
            pltpu.VMEM((16,), jnp.int32),
            pltpu.SemaphoreType.DMA,
            pltpu.SemaphoreType.DMA,
        ],
    )


def _sc_agg_body(g_hbm, elsrc_hbm, eldst_hbm, cnts_hbm, out_hbm,
                 acc, gbuf0, gbuf1, shb, lhb, cbuf, gsem0, gsem1, lsem):
    tg = _tile_id()
    zv = jnp.zeros((16,), jnp.float32)

    def za(i, _):
        for j in range(16):
            acc[pl.ds(i * 256 + j * 16, 16)] = zv
        return 0

    lax.fori_loop(0, _ACC_ROWS, za, 0)

    pltpu.sync_copy(cnts_hbm.at[tg], cbuf)
    cnt_pad = jnp.max(cbuf[pl.ds(0, 16)])

    bufs = (gbuf0, gbuf1)
    gsems = (gsem0, gsem1)
    lane = lax.iota(jnp.int32, 16)

    def gather(j, p):
        idx = shb.at[j // 2, pl.ds((j % 2) * 64, _CHUNK)]
        pltpu.async_copy(g_hbm.at[idx], bufs[p], gsems[p])

    def wait_g(p):
        idx = shb.at[0, pl.ds(0, _CHUNK)]
        pltpu.make_async_copy(g_hbm.at[idx], bufs[p], gsems[p]).wait()

    def accum(j, p):
        def pe(e, _):
            e16 = (e // 16) * 16
            lvec = lhb[j // 2, pl.ds((j % 2) * 64 + e16, 16)]
            msk = lane == (e % 16)
            b = jnp.max(jnp.where(msk, lvec * 256, 0))
            for f in range(16):
                acc[pl.ds(b + f * 16, 16)] = (
                    acc[pl.ds(b + f * 16, 16)]
                    + bufs[p][e, pl.ds(f * 16, 16)])
            return 0

        lax.fori_loop(0, _CHUNK, pe, 0)

    for h in range(2):
        pltpu.sync_copy(elsrc_hbm.at[tg, pl.ds(h * 32, 32)], shb)
        pltpu.sync_copy(eldst_hbm.at[tg, pl.ds(h * 32, 32)], lhb)
        trip = jnp.clip((cnt_pad - h * _HLIST) // _CHUNK, 0, _HLIST // _CHUNK)

        @pl.when(trip > 0)
        def _():
            gather(0, 0)

        def kb(k, _):
            j = 2 * k

            @pl.when(j < trip)
            def _():
                wait_g(0)

                @pl.when(j + 1 < trip)
                def _():
                    gather(j + 1, 1)

                accum(j, 0)

            @pl.when(j + 1 < trip)
            def _():
                wait_g(1)

                @pl.when(j + 2 < trip)
                def _():
                    gather(j + 2, 0)

                accum(j + 1, 1)

            return 0

        lax.fori_loop(0, _HLIST // _CHUNK // 2, kb, 0)

    nwords = _RNG * _D

    @pl.when(tg < _NW - 1)
    def _():
        pltpu.sync_copy(acc.at[pl.ds(0, nwords)],
                        out_hbm.at[pl.ds(tg * nwords, nwords)])

    last_words = (_N - (_NW - 1) * _RNG) * _D

    @pl.when(tg == _NW - 1)
    def _():
        pltpu.sync_copy(acc.at[pl.ds(0, last_words)],
                        out_hbm.at[pl.ds((_NW - 1) * nwords, last_words)])


def _make_sc_agg():
    return pl.kernel(
        _sc_agg_body,
        out_type=jax.ShapeDtypeStruct((_N * _D,), jnp.float32),
        mesh=_mesh,
        compiler_params=_sc_params,
        scratch_types=[
            pltpu.VMEM((_ACC_ROWS * _D,), jnp.float32),
            pltpu.VMEM((_CHUNK, _D), jnp.float32),
            pltpu.VMEM((_CHUNK, _D), jnp.float32),
            pltpu.VMEM((32, 128), jnp.int32),
            pltpu.VMEM((32, 128), jnp.int32),
            pltpu.VMEM((16,), jnp.int32),
            pltpu.SemaphoreType.DMA,
            pltpu.SemaphoreType.DMA,
            pltpu.SemaphoreType.DMA,
        ],
    )


def _sc_pool_body(h_hbm, batch_hbm, out_hbm, bbuf, hbuf, sacc, macc, obuf):
    tg = _tile_id()
    g0 = tg * 2
    pltpu.sync_copy(batch_hbm, bbuf)
    zi = jnp.zeros((16,), jnp.int32)

    def sb(k, cs):
        c0, c1, c2 = cs
        v = bbuf[pl.ds(k * 16, 16)]
        c0 = c0 + plsc.all_reduce_population_count(v < g0)
        c1 = c1 + plsc.all_reduce_population_count(v < g0 + 1)
        c2 = c2 + plsc.all_reduce_population_count(v < g0 + 2)
        return (c0, c1, c2)

    c0, c1, c2 = lax.fori_loop(0, _N // 16, sb, (zi, zi, zi))
    s0 = jnp.max(c0)
    s1 = jnp.max(c1)
    s2 = jnp.max(c2)
    zf = jnp.zeros((16,), jnp.float32)

    for gg, st, en in ((g0, s0, s1), (g0 + 1, s1, s2)):
        for j in range(16):
            sacc[pl.ds(j * 16, 16)] = zf
            macc[pl.ds(j * 16, 16)] = zf
        cnt = en - st
        st8 = (st // 8) * 8
        trips = (en - st8 + 15) // 16

        def cb(kc, _, st=st, en=en, st8=st8):
            r = st8 + kc * 16
            w0 = jnp.minimum(r, _N - 16)
            pltpu.sync_copy(h_hbm.at[pl.ds(w0, 16)], hbuf)
            for l in range(16):
                row = w0 + l
                cond = (row >= jnp.maximum(st, r)) & (row < jnp.minimum(
                    en, r + 16))
                for j in range(16):
                    v = hbuf[l, pl.ds(j * 16, 16)]
                    vm = jnp.where(cond, v, zf)
                    sacc[pl.ds(j * 16, 16)] = sacc[pl.ds(j * 16, 16)] + vm
                    macc[pl.ds(j * 16, 16)] = jnp.maximum(
                        macc[pl.ds(j * 16, 16)], vm)
            return 0

        lax.fori_loop(0, trips, cb, 0)
        cntv = jnp.full((16,), jnp.maximum(cnt, 1), jnp.int32).astype(
            jnp.float32)
        for j in range(16):
            obuf[pl.ds(j * 16, 16)] = sacc[pl.ds(j * 16, 16)] / cntv
            obuf[pl.ds(_D + j * 16, 16)] = macc[pl.ds(j * 16, 16)]
        pltpu.sync_copy(obuf, out_hbm.at[gg])


def _make_sc_pool():
    return pl.kernel(
        _sc_pool_body,
        out_type=jax.ShapeDtypeStruct((_G, 2 * _D), jnp.float32),
        mesh=_mesh,
        compiler_params=_sc_params,
        scratch_types=[
            pltpu.VMEM((_N,), jnp.int32),
            pltpu.VMEM((16, _D), jnp.float32),
            pltpu.VMEM((_D,), jnp.float32),
            pltpu.VMEM((_D,), jnp.float32),
            pltpu.VMEM((2 * _D,), jnp.float32),
        ],
    )


_R = 2000


def _tc_mm1_body(x_ref, w_ref, hist_ref, g_ref, dinv_ref):
    deg = jnp.sum(hist_ref[...], axis=1, keepdims=True) + 1.0
    dinv = lax.rsqrt(deg)
    dinv_ref[...] = dinv
    g_ref[...] = dinv * jnp.dot(
        x_ref[...], w_ref[...], preferred_element_type=jnp.float32)


_tc_mm1 = pl.pallas_call(
    _tc_mm1_body,
    grid=(_N // _R,),
    in_specs=[
        pl.BlockSpec((_R, _D), lambda i: (i, 0)),
        pl.BlockSpec((_D, _D), lambda i: (0, 0)),
        pl.BlockSpec((_R, 16), lambda i: (i, 0)),
    ],
    out_specs=[
        pl.BlockSpec((_R, _D), lambda i: (i, 0)),
        pl.BlockSpec((_R, 1), lambda i: (i, 0)),
    ],
    out_shape=[
        jax.ShapeDtypeStruct((_N, _D), jnp.float32),
        jax.ShapeDtypeStruct((_N, 1), jnp.float32),
    ],
)


def _tc_layer_body(agg_ref, g_ref, dinv_ref, b_ref, w_ref, o_ref):
    dinv = dinv_ref[...]
    h = jnp.maximum(dinv * (agg_ref[...] + g_ref[...]) + b_ref[...], 0.0)
    o_ref[...] = dinv * jnp.dot(
        h, w_ref[...], preferred_element_type=jnp.float32)


_tc_layer = pl.pallas_call(
    _tc_layer_body,
    grid=(_N // _R,),
    in_specs=[
        pl.BlockSpec((_R, _D), lambda i: (i, 0)),
        pl.BlockSpec((_R, _D), lambda i: (i, 0)),
        pl.BlockSpec((_R, 1), lambda i: (i, 0)),
        pl.BlockSpec((1, _D), lambda i: (0, 0)),
        pl.BlockSpec((_D, _D), lambda i: (0, 0)),
    ],
    out_specs=pl.BlockSpec((_R, _D), lambda i: (i, 0)),
    out_shape=jax.ShapeDtypeStruct((_N, _D), jnp.float32),
)


def _tc_epi_body(agg_ref, g_ref, dinv_ref, b_ref, o_ref):
    o_ref[...] = jnp.maximum(
        dinv_ref[...] * (agg_ref[...] + g_ref[...]) + b_ref[...], 0.0)


_tc_epi = pl.pallas_call(
    _tc_epi_body,
    grid=(_N // _R,),
    in_specs=[
        pl.BlockSpec((_R, _D), lambda i: (i, 0)),
        pl.BlockSpec((_R, _D), lambda i: (i, 0)),
        pl.BlockSpec((_R, 1), lambda i: (i, 0)),
        pl.BlockSpec((1, _D), lambda i: (0, 0)),
    ],
    out_specs=pl.BlockSpec((_R, _D), lambda i: (i, 0)),
    out_shape=jax.ShapeDtypeStruct((_N, _D), jnp.float32),
)


def kernel(x, edge_index, edge_attr, batch, W1, b1, W2, b2, W3, b3):
    del edge_attr
    src = edge_index[0]
    dst = edge_index[1]
    pad = _EP - _E
    src128 = jnp.concatenate(
        [src, jnp.zeros((pad,), jnp.int32)]).reshape(_ER, 128)
    dst128 = jnp.concatenate(
        [dst, jnp.full((pad,), _EPAD_DST, jnp.int32)]).reshape(_ER, 128)

    sc_prep = _make_sc_prep()
    sc_agg = _make_sc_agg()
    sc_pool = _make_sc_pool()

    elsrc, eldst, cnts, hist = sc_prep(src128, dst128)
    elsrc = elsrc.reshape(_NW, _CAP // 128, 128)
    eldst = eldst.reshape(_NW, _CAP // 128, 128)
    hist2 = hist.reshape(_NW * 40 * 128 // 16, 16)[:_N]

    g1, dinv = _tc_mm1(x, W1, hist2)
    agg1 = sc_agg(g1, elsrc, eldst, cnts).reshape(_N, _D)
    g2 = _tc_layer(agg1, g1, dinv, b1.reshape(1, _D), W2)
    agg2 = sc_agg(g2, elsrc, eldst, cnts).reshape(_N, _D)
    g3 = _tc_layer(agg2, g2, dinv, b2.reshape(1, _D), W3)
    agg3 = sc_agg(g3, elsrc, eldst, cnts).reshape(_N, _D)
    h3 = _tc_epi(agg3, g3, dinv, b3.reshape(1, _D))
    return sc_pool(h3, batch)
